# baseline (speedup 1.0000x reference)
import jax
import jax.numpy as jnp
from jax import lax
from jax.experimental import pallas as pl
from jax.experimental.pallas import tpu as pltpu
from jax.experimental.pallas import tpu_sc as plsc

_EXTRA = (3216, 3226, 3387, 6617, 6624, 6787,
          2746, 2319, 2445, 2556, 2673,
          6191, 5782, 5905, 6016, 6133)

_B, _V, _J, _E, _C = 4096, 6890, 52, 16, 3
_NC, _NS = 2, 16
_NW = _NC * _NS
_CB = _B // _NW
_L = 16


def _body(vt, jt, out, gbuf, obuf, sem):
  wid = lax.axis_index("s") * _NC + lax.axis_index("c")
  cols = pl.ds(wid * _CB, _CB)

  copies = [pltpu.make_async_copy(
      jt.at[:, :, cols], obuf.at[:, pl.ds(0, _J), :], sem)]
  for j, vidx in enumerate(_EXTRA):
    g0 = (vidx // 8) * 8
    copies.append(pltpu.make_async_copy(
        vt.at[:, pl.ds(g0, 8), cols], gbuf.at[j], sem))
  for cp in copies:
    cp.start()

  for j, vidx in enumerate(_EXTRA):
    copies[1 + j].wait()

    def extract(k, carry, j=j, vidx=vidx):
      off = pl.multiple_of(k * _L, _L)
      for c in range(_C):
        obuf[c, _J + j, pl.ds(off, _L)] = gbuf[j, c, vidx % 8, pl.ds(off, _L)]
      return carry

    lax.fori_loop(0, _CB // _L, extract, 0)

  copies[0].wait()
  pltpu.sync_copy(obuf, out.at[:, :, cols])


@jax.jit
def kernel(vertices, joints):
  vt = jnp.transpose(vertices, (2, 1, 0))
  jt = jnp.transpose(joints, (2, 1, 0))
  mesh = plsc.VectorSubcoreMesh(core_axis_name="c", subcore_axis_name="s")
  out_t = pl.kernel(
      _body,
      out_type=jax.ShapeDtypeStruct((_C, _J + _E, _B), jnp.float32),
      mesh=mesh,
      compiler_params=pltpu.CompilerParams(
          disable_bounds_checks=True,
          disable_semaphore_checks=True,
          skip_device_barrier=True,
      ),
      scratch_types=[
          pltpu.VMEM((_E, _C, 8, _CB), jnp.float32),
          pltpu.VMEM((_C, _J + _E, _CB), jnp.float32),
          pltpu.SemaphoreType.DMA,
      ],
  )(vt, jt)
  return jnp.transpose(out_t, (2, 1, 0))


# speedup vs baseline: 1.0189x; 1.0189x over previous
import jax
import jax.numpy as jnp
from jax import lax
from jax.experimental import pallas as pl
from jax.experimental.pallas import tpu as pltpu
from jax.experimental.pallas import tpu_sc as plsc

_EXTRA = (3216, 3226, 3387, 6617, 6624, 6787,
          2746, 2319, 2445, 2556, 2673,
          6191, 5782, 5905, 6016, 6133)

_B, _V, _J, _E, _C = 4096, 6890, 52, 16, 3
_NC, _NS = 2, 16
_NW = _NC * _NS
_CB = _B // _NW
_L = 16


def _body(vt, jt, out, gbuf, obuf, sem):
  wid = lax.axis_index("s") * _NC + lax.axis_index("c")
  cols = pl.ds(wid * _CB, _CB)

  copies = [pltpu.make_async_copy(
      jt.at[:, :, cols], obuf.at[:, pl.ds(0, _J), :], sem)]
  for j, vidx in enumerate(_EXTRA):
    g0 = (vidx // 8) * 8
    copies.append(pltpu.make_async_copy(
        vt.at[:, pl.ds(g0, 8), cols], gbuf.at[j], sem))
  for cp in copies:
    cp.start()
  for cp in copies:
    cp.wait()

  def extract(k, carry):
    off = pl.multiple_of(k * _L, _L)
    for j, vidx in enumerate(_EXTRA):
      for c in range(_C):
        obuf[c, _J + j, pl.ds(off, _L)] = gbuf[j, c, vidx % 8, pl.ds(off, _L)]
    return carry

  lax.fori_loop(0, _CB // _L, extract, 0)

  pltpu.sync_copy(obuf, out.at[:, :, cols])


@jax.jit
def kernel(vertices, joints):
  vt = jnp.transpose(vertices, (2, 1, 0))
  jt = jnp.transpose(joints, (2, 1, 0))
  mesh = plsc.VectorSubcoreMesh(core_axis_name="c", subcore_axis_name="s")
  out_t = pl.kernel(
      _body,
      out_type=jax.ShapeDtypeStruct((_C, _J + _E, _B), jnp.float32),
      mesh=mesh,
      scratch_types=[
          pltpu.VMEM((_E, _C, 8, _CB), jnp.float32),
          pltpu.VMEM((_C, _J + _E, _CB), jnp.float32),
          pltpu.SemaphoreType.DMA,
      ],
  )(vt, jt)
  return jnp.transpose(out_t, (2, 1, 0))
